# Initial kernel scaffold; baseline (speedup 1.0000x reference)
import jax
import jax.numpy as jnp
from jax.experimental import pallas as pl


def kernel(x, edge_index, W1, b1, Wl, bl, Wr, W2, b2):
    raise NotImplementedError("write your pallas kernel here")


# speedup vs baseline: 14.6734x; 14.6734x over previous
import jax
import jax.numpy as jnp
from jax import lax
from jax.experimental import pallas as pl
from jax.experimental.pallas import tpu as pltpu
from jax.experimental.pallas import tpu_sc as plsc

N = 100000
D = 16
H = 32
E = 1600000
NC = 2
NS = 16
NW = NC * NS
EW = E // NW
CHUNK = 1000
NCHUNK = EW // CHUNK
ROWS_T = 6256
_SUBCHUNKS = ((0, 1000), (1000, 1000), (2000, 1000), (3000, 1000),
              (4000, 1000), (5000, 1000), (6000, 256))
BLK = 2000


def _lin1_body(x_ref, w_ref, b_ref, o_ref):
    o_ref[...] = jnp.maximum(
        jnp.dot(x_ref[...], w_ref[...], preferred_element_type=jnp.float32)
        + b_ref[...], 0.0)


def _lin1(x, W1, b1):
    return pl.pallas_call(
        _lin1_body,
        grid=(N // BLK,),
        in_specs=[pl.BlockSpec((BLK, D), lambda i: (i, 0)),
                  pl.BlockSpec((D, D), lambda i: (0, 0)),
                  pl.BlockSpec((1, D), lambda i: (0, 0))],
        out_specs=pl.BlockSpec((BLK, D), lambda i: (i, 0)),
        out_shape=jax.ShapeDtypeStruct((N, D), jnp.float32),
    )(x, W1, b1.reshape(1, D))


def _sc_body(h_hbm, src_hbm, dst_hbm, ones_hbm, z2d_hbm, z1d_hbm,
             sum_out, cnt_out,
             acc_sh, cnt_sh, src_v, dst_v, rows_v, ones_v, cb_v, sem):
    c = lax.axis_index("c")
    s = lax.axis_index("s")
    zstart = jnp.minimum(s * ROWS_T, N - ROWS_T)
    pltpu.sync_copy(z2d_hbm, acc_sh.at[pl.ds(zstart, ROWS_T)])
    pltpu.sync_copy(z1d_hbm, cb_v)
    for off, ln in _SUBCHUNKS:
        pltpu.sync_copy(cb_v.at[pl.ds(0, ln)], cnt_sh.at[pl.ds(zstart + off, ln)])
    pltpu.sync_copy(ones_hbm, ones_v)
    plsc.subcore_barrier()

    base = (c * NS + s) * EW

    def chunk(i, carry):
        off = base + i * CHUNK
        pltpu.sync_copy(src_hbm.at[pl.ds(off, CHUNK)], src_v)
        pltpu.sync_copy(dst_hbm.at[pl.ds(off, CHUNK)], dst_v)
        pltpu.async_copy(h_hbm.at[src_v], rows_v, sem).wait()
        pltpu.sync_copy(rows_v, acc_sh.at[dst_v], add=True)
        pltpu.sync_copy(ones_v, cnt_sh.at[dst_v], add=True)
        return carry

    lax.fori_loop(0, NCHUNK, chunk, 0)
    plsc.subcore_barrier()
    pltpu.sync_copy(acc_sh.at[pl.ds(zstart, ROWS_T)],
                    sum_out.at[c, pl.ds(zstart, ROWS_T)])
    for off, ln in _SUBCHUNKS:
        pltpu.sync_copy(cnt_sh.at[pl.ds(zstart + off, ln)], cb_v.at[pl.ds(0, ln)])
        pltpu.sync_copy(cb_v.at[pl.ds(0, ln)],
                        cnt_out.at[pl.ds(c * N + zstart + off, ln)])


def _sc_aggregate(h, src, dst):
    ones = jnp.ones((CHUNK,), jnp.float32)
    z2d = jnp.zeros((ROWS_T, D), jnp.float32)
    z1d = jnp.zeros((CHUNK,), jnp.float32)
    mesh = plsc.VectorSubcoreMesh(core_axis_name="c", subcore_axis_name="s")
    f = pl.kernel(
        _sc_body,
        out_type=[jax.ShapeDtypeStruct((NC, N, D), jnp.float32),
                  jax.ShapeDtypeStruct((NC * N,), jnp.float32)],
        mesh=mesh,
        scratch_types=[
            pltpu.VMEM_SHARED((N, D), jnp.float32),
            pltpu.VMEM_SHARED((N,), jnp.float32),
            pltpu.VMEM((CHUNK,), jnp.int32),
            pltpu.VMEM((CHUNK,), jnp.int32),
            pltpu.VMEM((CHUNK, D), jnp.float32),
            pltpu.VMEM((CHUNK,), jnp.float32),
            pltpu.VMEM((CHUNK,), jnp.float32),
            pltpu.SemaphoreType.DMA,
        ],
        compiler_params=pltpu.CompilerParams(use_tc_tiling_on_sc=False),
    )
    return f(h, src, dst, ones, z2d, z1d)


def _combine_body(h_ref, s0_ref, s1_ref, c0_ref, c1_ref,
                  wl_ref, bl_ref, wr_ref, w2_ref, b2_ref, o_ref):
    cnt = jnp.maximum(c0_ref[...] + c1_ref[...], 1.0)
    agg = (s0_ref[...] + s1_ref[...]) / cnt
    h2 = jnp.maximum(
        jnp.dot(agg, wl_ref[...], preferred_element_type=jnp.float32)
        + bl_ref[...]
        + jnp.dot(h_ref[...], wr_ref[...], preferred_element_type=jnp.float32),
        0.0)
    o_ref[...] = (jnp.dot(h2, w2_ref[...], preferred_element_type=jnp.float32)
                  + b2_ref[...])


def _combine(h, s0, s1, c0, c1, Wl, bl, Wr, W2, b2):
    return pl.pallas_call(
        _combine_body,
        grid=(N // BLK,),
        in_specs=[pl.BlockSpec((BLK, D), lambda i: (i, 0)),
                  pl.BlockSpec((BLK, D), lambda i: (i, 0)),
                  pl.BlockSpec((BLK, D), lambda i: (i, 0)),
                  pl.BlockSpec((BLK, 1), lambda i: (i, 0)),
                  pl.BlockSpec((BLK, 1), lambda i: (i, 0)),
                  pl.BlockSpec((D, H), lambda i: (0, 0)),
                  pl.BlockSpec((1, H), lambda i: (0, 0)),
                  pl.BlockSpec((D, H), lambda i: (0, 0)),
                  pl.BlockSpec((H, H), lambda i: (0, 0)),
                  pl.BlockSpec((1, H), lambda i: (0, 0))],
        out_specs=pl.BlockSpec((BLK, H), lambda i: (i, 0)),
        out_shape=jax.ShapeDtypeStruct((N, H), jnp.float32),
    )(h, s0, s1, c0, c1, Wl, bl.reshape(1, H), Wr, W2, b2.reshape(1, H))


def kernel(x, edge_index, W1, b1, Wl, bl, Wr, W2, b2):
    ei = edge_index.astype(jnp.int32)
    src = ei[0]
    dst = ei[1]
    h = _lin1(x, W1, b1)
    summed, cnt = _sc_aggregate(h, src, dst)
    return _combine(h, summed[0], summed[1],
                    cnt[:N].reshape(N, 1), cnt[N:].reshape(N, 1),
                    Wl, bl, Wr, W2, b2)


# speedup vs baseline: 16.1970x; 1.1038x over previous
import jax
import jax.numpy as jnp
from jax import lax
from jax.experimental import pallas as pl
from jax.experimental.pallas import tpu as pltpu
from jax.experimental.pallas import tpu_sc as plsc

N = 100000
NP = 102400
PR = NP // 8
D = 16
H = 32
E = 1600000
NC = 2
NS = 16
EW = E // (NC * NS)
CHUNK = 1000
NCHUNK = EW // CHUNK
ROWS_T = NP // NS
BLKP = 512


def _lin1p_body(x_ref, w_ref, b_ref, o_ref):
    o_ref[...] = jnp.maximum(
        jnp.dot(x_ref[...], w_ref[...], preferred_element_type=jnp.float32)
        + b_ref[...], 0.0)


def _lin1p(xp, W1b, b1b):
    return pl.pallas_call(
        _lin1p_body,
        grid=(PR // BLKP,),
        in_specs=[pl.BlockSpec((BLKP, 128), lambda i: (i, 0)),
                  pl.BlockSpec((128, 128), lambda i: (0, 0)),
                  pl.BlockSpec((1, 128), lambda i: (0, 0))],
        out_specs=pl.BlockSpec((BLKP, 128), lambda i: (i, 0)),
        out_shape=jax.ShapeDtypeStruct((PR, 128), jnp.float32),
    )(xp, W1b, b1b)


def _sc_body(h_hbm, src_hbm, dst_hbm, ones_hbm, z2d_hbm, z1d_hbm,
             sum_out, cnt_out,
             acc_sh, cnt_sh, src_v, dst_v, rows_v, ones_v, cb_v, sem):
    c = lax.axis_index("c")
    s = lax.axis_index("s")
    zstart = s * ROWS_T
    pltpu.sync_copy(z2d_hbm, acc_sh.at[pl.ds(zstart, ROWS_T)])
    pltpu.sync_copy(z1d_hbm, cb_v)
    for off, ln in _SUBCHUNKS:
        pltpu.sync_copy(cb_v.at[pl.ds(0, ln)], cnt_sh.at[pl.ds(zstart + off, ln)])
    pltpu.sync_copy(ones_hbm, ones_v)
    plsc.subcore_barrier()

    base = (c * NS + s) * EW

    def chunk(i, carry):
        off = base + i * CHUNK
        pltpu.sync_copy(src_hbm.at[pl.ds(off, CHUNK)], src_v)
        pltpu.sync_copy(dst_hbm.at[pl.ds(off, CHUNK)], dst_v)
        pltpu.async_copy(h_hbm.at[src_v], rows_v, sem).wait()
        pltpu.sync_copy(rows_v, acc_sh.at[dst_v], add=True)
        pltpu.sync_copy(ones_v, cnt_sh.at[dst_v], add=True)
        return carry

    lax.fori_loop(0, NCHUNK, chunk, 0)
    plsc.subcore_barrier()
    pltpu.sync_copy(acc_sh.at[pl.ds(zstart, ROWS_T)],
                    sum_out.at[c, pl.ds(zstart, ROWS_T)])
    for off, ln in _SUBCHUNKS:
        pltpu.sync_copy(cnt_sh.at[pl.ds(zstart + off, ln)], cb_v.at[pl.ds(0, ln)])
        pltpu.sync_copy(cb_v.at[pl.ds(0, ln)],
                        cnt_out.at[pl.ds(c * NP + zstart + off, ln)])


_SUBCHUNKS = tuple((k * CHUNK, CHUNK) for k in range(ROWS_T // CHUNK)) + (
    ((ROWS_T // CHUNK) * CHUNK, ROWS_T % CHUNK),)
_SUBCHUNKS = tuple((o, ln) for o, ln in _SUBCHUNKS if ln)


def _sc_aggregate(h_lin, src, dst):
    ones = jnp.ones((CHUNK,), jnp.float32)
    z2d = jnp.zeros((ROWS_T, D), jnp.float32)
    z1d = jnp.zeros((CHUNK,), jnp.float32)
    mesh = plsc.VectorSubcoreMesh(core_axis_name="c", subcore_axis_name="s")
    f = pl.kernel(
        _sc_body,
        out_type=[jax.ShapeDtypeStruct((NC, NP, D), jnp.float32),
                  jax.ShapeDtypeStruct((NC * NP,), jnp.float32)],
        mesh=mesh,
        scratch_types=[
            pltpu.VMEM_SHARED((NP, D), jnp.float32),
            pltpu.VMEM_SHARED((NP,), jnp.float32),
            pltpu.VMEM((CHUNK,), jnp.int32),
            pltpu.VMEM((CHUNK,), jnp.int32),
            pltpu.VMEM((CHUNK, D), jnp.float32),
            pltpu.VMEM((CHUNK,), jnp.float32),
            pltpu.VMEM((CHUNK,), jnp.float32),
            pltpu.SemaphoreType.DMA,
        ],
        compiler_params=pltpu.CompilerParams(use_tc_tiling_on_sc=False),
    )
    return f(h_lin, src, dst, ones, z2d, z1d)


def _combinep_body(h_ref, s0_ref, s1_ref, d_ref,
                   wl_ref, bl_ref, wr_ref, w2_ref, b2_ref, o_ref):
    agg = (s0_ref[...] + s1_ref[...]) / jnp.maximum(d_ref[...], 1.0)
    h2 = jnp.maximum(
        jnp.dot(agg, wl_ref[...], preferred_element_type=jnp.float32)
        + bl_ref[...]
        + jnp.dot(h_ref[...], wr_ref[...], preferred_element_type=jnp.float32),
        0.0)
    o_ref[...] = (jnp.dot(h2, w2_ref[...], preferred_element_type=jnp.float32)
                  + b2_ref[...])


def _combinep(hp, s0p, s1p, divp, Wlb, blb, Wrb, W2b, b2b):
    return pl.pallas_call(
        _combinep_body,
        grid=(PR // BLKP,),
        in_specs=[pl.BlockSpec((BLKP, 128), lambda i: (i, 0)),
                  pl.BlockSpec((BLKP, 128), lambda i: (i, 0)),
                  pl.BlockSpec((BLKP, 128), lambda i: (i, 0)),
                  pl.BlockSpec((BLKP, 128), lambda i: (i, 0)),
                  pl.BlockSpec((128, 256), lambda i: (0, 0)),
                  pl.BlockSpec((1, 256), lambda i: (0, 0)),
                  pl.BlockSpec((128, 256), lambda i: (0, 0)),
                  pl.BlockSpec((256, 256), lambda i: (0, 0)),
                  pl.BlockSpec((1, 256), lambda i: (0, 0))],
        out_specs=pl.BlockSpec((BLKP, 256), lambda i: (i, 0)),
        out_shape=jax.ShapeDtypeStruct((PR, 256), jnp.float32),
    )(hp, s0p, s1p, divp, Wlb, blb, Wrb, W2b, b2b)


def _block_diag8(W):
    a, b = W.shape
    eye = jnp.eye(8, dtype=W.dtype)
    return (eye[:, None, :, None] * W[None, :, None, :]).reshape(8 * a, 8 * b)


def kernel(x, edge_index, W1, b1, Wl, bl, Wr, W2, b2):
    ei = edge_index.astype(jnp.int32)
    src = ei[0]
    dst = ei[1]

    W1b = _block_diag8(W1)
    b1b = jnp.tile(b1, 8).reshape(1, 128)
    Wlb = _block_diag8(Wl)
    blb = jnp.tile(bl, 8).reshape(1, 256)
    Wrb = _block_diag8(Wr)
    W2b = _block_diag8(W2)
    b2b = jnp.tile(b2, 8).reshape(1, 256)

    xp = jnp.pad(x.reshape(N // 8, 128), ((0, PR - N // 8), (0, 0)))
    hp = _lin1p(xp, W1b, b1b)
    summed, cnt = _sc_aggregate(hp.reshape(NP, D), src, dst)
    s0p = summed[0].reshape(PR, 128)
    s1p = summed[1].reshape(PR, 128)
    cm = cnt[:NP] + cnt[NP:]
    divp = jnp.repeat(cm.reshape(PR, 8), D, axis=1)
    outp = _combinep(hp, s0p, s1p, divp, Wlb, blb, Wrb, W2b, b2b)
    return outp.reshape(NP, H)[:N]


# speedup vs baseline: 23.4523x; 1.4479x over previous
import jax
import jax.numpy as jnp
from jax import lax
from jax.experimental import pallas as pl
from jax.experimental.pallas import tpu as pltpu
from jax.experimental.pallas import tpu_sc as plsc

N = 100000
NP = 102400
PR = NP // 8
D = 16
H = 32
E = 1600000
NC = 2
NS = 16
EW = E // (NC * NS)
CHUNK = 1000
NCHUNK = EW // CHUNK
ROWS_T = NP // NS
BLKP = 512


def _lin1p_body(x_ref, w_ref, b_ref, o_ref):
    o_ref[...] = jnp.maximum(
        jnp.dot(x_ref[...], w_ref[...], preferred_element_type=jnp.float32)
        + b_ref[...], 0.0)


def _lin1p(xp, W1b, b1b):
    return pl.pallas_call(
        _lin1p_body,
        grid=(PR // BLKP,),
        in_specs=[pl.BlockSpec((BLKP, 128), lambda i: (i, 0)),
                  pl.BlockSpec((128, 128), lambda i: (0, 0)),
                  pl.BlockSpec((1, 128), lambda i: (0, 0))],
        out_specs=pl.BlockSpec((BLKP, 128), lambda i: (i, 0)),
        out_shape=jax.ShapeDtypeStruct((PR, 128), jnp.float32),
    )(xp, W1b, b1b)


def _sc_body(h_hbm, src_hbm, dst_hbm, ones_hbm, z2d_hbm, z1d_hbm,
             sum_out, cnt_out,
             acc_sh, cnt_sh, src_v, dst_v, rows_v, ones_v, cb_v, sem):
    c = lax.axis_index("c")
    s = lax.axis_index("s")
    zstart = s * ROWS_T
    pltpu.sync_copy(z2d_hbm, acc_sh.at[pl.ds(zstart, ROWS_T)])
    pltpu.sync_copy(z1d_hbm, cb_v)
    for off, ln in _SUBCHUNKS:
        pltpu.sync_copy(cb_v.at[pl.ds(0, ln)], cnt_sh.at[pl.ds(zstart + off, ln)])
    pltpu.sync_copy(ones_hbm, ones_v)
    plsc.subcore_barrier()

    base = (c * NS + s) * EW

    def chunk(i, carry):
        off = base + i * CHUNK
        pltpu.sync_copy(src_hbm.at[pl.ds(off, CHUNK)], src_v)
        pltpu.sync_copy(dst_hbm.at[pl.ds(off, CHUNK)], dst_v)
        pltpu.async_copy(h_hbm.at[src_v], rows_v, sem).wait()
        pltpu.sync_copy(rows_v, acc_sh.at[dst_v], add=True)
        pltpu.sync_copy(ones_v, cnt_sh.at[dst_v], add=True)
        return carry

    lax.fori_loop(0, NCHUNK, chunk, 0)
    plsc.subcore_barrier()
    pltpu.sync_copy(acc_sh.at[pl.ds(zstart, ROWS_T)],
                    sum_out.at[c, pl.ds(zstart, ROWS_T)])
    for off, ln in _SUBCHUNKS:
        pltpu.sync_copy(cnt_sh.at[pl.ds(zstart + off, ln)], cb_v.at[pl.ds(0, ln)])
        pltpu.sync_copy(cb_v.at[pl.ds(0, ln)],
                        cnt_out.at[pl.ds(c * NP + zstart + off, ln)])


_SUBCHUNKS = tuple((k * CHUNK, CHUNK) for k in range(ROWS_T // CHUNK)) + (
    ((ROWS_T // CHUNK) * CHUNK, ROWS_T % CHUNK),)
_SUBCHUNKS = tuple((o, ln) for o, ln in _SUBCHUNKS if ln)


def _sc_aggregate(h_lin, src, dst):
    ones = jnp.ones((CHUNK,), jnp.float32)
    z2d = jnp.zeros((ROWS_T, D), jnp.float32)
    z1d = jnp.zeros((CHUNK,), jnp.float32)
    mesh = plsc.VectorSubcoreMesh(core_axis_name="c", subcore_axis_name="s")
    f = pl.kernel(
        _sc_body,
        out_type=[jax.ShapeDtypeStruct((NC, NP, D), jnp.float32),
                  jax.ShapeDtypeStruct((NC * NP,), jnp.float32)],
        mesh=mesh,
        scratch_types=[
            pltpu.VMEM_SHARED((NP, D), jnp.float32),
            pltpu.VMEM_SHARED((NP,), jnp.float32),
            pltpu.VMEM((CHUNK,), jnp.int32),
            pltpu.VMEM((CHUNK,), jnp.int32),
            pltpu.VMEM((CHUNK, D), jnp.float32),
            pltpu.VMEM((CHUNK,), jnp.float32),
            pltpu.VMEM((CHUNK,), jnp.float32),
            pltpu.SemaphoreType.DMA,
        ],
        compiler_params=pltpu.CompilerParams(use_tc_tiling_on_sc=False),
    )
    return f(h_lin, src, dst, ones, z2d, z1d)


def _combinep_body(h_ref, sp_ref, d_ref,
                   wl_ref, bl_ref, wr_ref, w2_ref, b2_ref, o_ref):
    agg = (sp_ref[0] + sp_ref[1]) / jnp.maximum(d_ref[...], 1.0)
    h2 = jnp.maximum(
        jnp.dot(agg, wl_ref[...], preferred_element_type=jnp.float32)
        + bl_ref[...]
        + jnp.dot(h_ref[...], wr_ref[...], preferred_element_type=jnp.float32),
        0.0)
    o_ref[...] = (jnp.dot(h2, w2_ref[...], preferred_element_type=jnp.float32)
                  + b2_ref[...])


def _combinep(hp, sp, divp, Wlb, blb, Wrb, W2b, b2b):
    return pl.pallas_call(
        _combinep_body,
        grid=(PR // BLKP,),
        in_specs=[pl.BlockSpec((BLKP, 128), lambda i: (i, 0)),
                  pl.BlockSpec((NC, BLKP, 128), lambda i: (0, i, 0)),
                  pl.BlockSpec((BLKP, 128), lambda i: (i, 0)),
                  pl.BlockSpec((128, 256), lambda i: (0, 0)),
                  pl.BlockSpec((1, 256), lambda i: (0, 0)),
                  pl.BlockSpec((128, 256), lambda i: (0, 0)),
                  pl.BlockSpec((256, 256), lambda i: (0, 0)),
                  pl.BlockSpec((1, 256), lambda i: (0, 0))],
        out_specs=pl.BlockSpec((BLKP, 256), lambda i: (i, 0)),
        out_shape=jax.ShapeDtypeStruct((PR, 256), jnp.float32),
    )(hp, sp, divp, Wlb, blb, Wrb, W2b, b2b)


def _block_diag8(W):
    a, b = W.shape
    eye = jnp.eye(8, dtype=W.dtype)
    return (eye[:, None, :, None] * W[None, :, None, :]).reshape(8 * a, 8 * b)


def kernel(x, edge_index, W1, b1, Wl, bl, Wr, W2, b2):
    ei = edge_index.astype(jnp.int32)
    src = ei[0]
    dst = ei[1]

    W1b = _block_diag8(W1)
    b1b = jnp.tile(b1, 8).reshape(1, 128)
    Wlb = _block_diag8(Wl)
    blb = jnp.tile(bl, 8).reshape(1, 256)
    Wrb = _block_diag8(Wr)
    W2b = _block_diag8(W2)
    b2b = jnp.tile(b2, 8).reshape(1, 256)

    xp = jnp.pad(x.reshape(N // 8, 128), ((0, PR - N // 8), (0, 0)))
    hp = _lin1p(xp, W1b, b1b)
    summed, cnt = _sc_aggregate(hp.reshape(NP, D), src, dst)
    sp = summed.reshape(NC, PR, 128)
    cm = cnt[:NP] + cnt[NP:]
    expand = jnp.kron(jnp.eye(8, dtype=jnp.float32), jnp.ones((1, D), jnp.float32))
    divp = cm.reshape(PR, 8) @ expand
    outp = _combinep(hp, sp, divp, Wlb, blb, Wrb, W2b, b2b)
    return outp.reshape(NP, H)[:N]


# speedup vs baseline: 23.6677x; 1.0092x over previous
import jax
import jax.numpy as jnp
from jax import lax
from jax.experimental import pallas as pl
from jax.experimental.pallas import tpu as pltpu
from jax.experimental.pallas import tpu_sc as plsc

N = 100000
NP = 102400
PR = NP // 8
D = 16
H = 32
E = 1600000
NC = 2
NS = 16
EW = E // (NC * NS)
CHUNK = 400
NCHUNK = EW // CHUNK
ROWS_T = NP // NS
BLKP = 512


def _lin1p_body(x_ref, w_ref, b_ref, o_ref):
    o_ref[...] = jnp.maximum(
        jnp.dot(x_ref[...], w_ref[...], preferred_element_type=jnp.float32)
        + b_ref[...], 0.0)


def _lin1p(xp, W1b, b1b):
    return pl.pallas_call(
        _lin1p_body,
        grid=(PR // BLKP,),
        in_specs=[pl.BlockSpec((BLKP, 128), lambda i: (i, 0)),
                  pl.BlockSpec((128, 128), lambda i: (0, 0)),
                  pl.BlockSpec((1, 128), lambda i: (0, 0))],
        out_specs=pl.BlockSpec((BLKP, 128), lambda i: (i, 0)),
        out_shape=jax.ShapeDtypeStruct((PR, 128), jnp.float32),
    )(xp, W1b, b1b)


def _sc_body(h_hbm, src_hbm, dst_hbm, ones_hbm, z2d_hbm, z1d_hbm,
             sum_out, cnt_out,
             acc_sh, cnt_sh, src_v0, dst_v0, rows_v0, src_v1, dst_v1, rows_v1,
             ones_v, cb_v, sem0, sem1):
    c = lax.axis_index("c")
    s = lax.axis_index("s")
    zstart = s * ROWS_T
    pltpu.sync_copy(z2d_hbm, acc_sh.at[pl.ds(zstart, ROWS_T)])
    pltpu.sync_copy(z1d_hbm, cb_v)
    for off, ln in _SUBCHUNKS:
        pltpu.sync_copy(cb_v.at[pl.ds(0, ln)], cnt_sh.at[pl.ds(zstart + off, ln)])
    pltpu.sync_copy(ones_hbm, ones_v)
    plsc.subcore_barrier()

    base = (c * NS + s) * EW

    def issue(off, src_v, dst_v, rows_v, sem):
        pltpu.sync_copy(src_hbm.at[pl.ds(off, CHUNK)], src_v)
        pltpu.sync_copy(dst_hbm.at[pl.ds(off, CHUNK)], dst_v)
        pltpu.async_copy(h_hbm.at[src_v], rows_v, sem)

    def drain_scatter(src_v, dst_v, rows_v, sem):
        pltpu.make_async_copy(h_hbm.at[src_v], rows_v, sem).wait()
        pltpu.sync_copy(rows_v, acc_sh.at[dst_v], add=True)
        pltpu.sync_copy(ones_v, cnt_sh.at[dst_v], add=True)

    issue(base, src_v0, dst_v0, rows_v0, sem0)

    def pair(j, carry):
        off = base + 2 * j * CHUNK
        issue(off + CHUNK, src_v1, dst_v1, rows_v1, sem1)
        drain_scatter(src_v0, dst_v0, rows_v0, sem0)
        issue(off + 2 * CHUNK, src_v0, dst_v0, rows_v0, sem0)
        drain_scatter(src_v1, dst_v1, rows_v1, sem1)
        return carry

    lax.fori_loop(0, (NCHUNK - 1) // 2, pair, 0)
    drain_scatter(src_v0, dst_v0, rows_v0, sem0)
    plsc.subcore_barrier()
    pltpu.sync_copy(acc_sh.at[pl.ds(zstart, ROWS_T)],
                    sum_out.at[c, pl.ds(zstart, ROWS_T)])
    for off, ln in _SUBCHUNKS:
        pltpu.sync_copy(cnt_sh.at[pl.ds(zstart + off, ln)], cb_v.at[pl.ds(0, ln)])
        pltpu.sync_copy(cb_v.at[pl.ds(0, ln)],
                        cnt_out.at[pl.ds(c * NP + zstart + off, ln)])


CSTAGE = 800
_SUBCHUNKS = tuple((k * CSTAGE, CSTAGE) for k in range(ROWS_T // CSTAGE))


def _sc_aggregate(h_lin, src, dst):
    ones = jnp.ones((CHUNK,), jnp.float32)
    z2d = jnp.zeros((ROWS_T, D), jnp.float32)
    z1d = jnp.zeros((CSTAGE,), jnp.float32)
    mesh = plsc.VectorSubcoreMesh(core_axis_name="c", subcore_axis_name="s")
    f = pl.kernel(
        _sc_body,
        out_type=[jax.ShapeDtypeStruct((NC, NP, D), jnp.float32),
                  jax.ShapeDtypeStruct((NC * NP,), jnp.float32)],
        mesh=mesh,
        scratch_types=[
            pltpu.VMEM_SHARED((NP, D), jnp.float32),
            pltpu.VMEM_SHARED((NP,), jnp.float32),
            pltpu.VMEM((CHUNK,), jnp.int32),
            pltpu.VMEM((CHUNK,), jnp.int32),
            pltpu.VMEM((CHUNK, D), jnp.float32),
            pltpu.VMEM((CHUNK,), jnp.int32),
            pltpu.VMEM((CHUNK,), jnp.int32),
            pltpu.VMEM((CHUNK, D), jnp.float32),
            pltpu.VMEM((CHUNK,), jnp.float32),
            pltpu.VMEM((CSTAGE,), jnp.float32),
            pltpu.SemaphoreType.DMA,
            pltpu.SemaphoreType.DMA,
        ],
        compiler_params=pltpu.CompilerParams(use_tc_tiling_on_sc=False),
    )
    return f(h_lin, src, dst, ones, z2d, z1d)


def _combinep_body(h_ref, sp_ref, d_ref,
                   wl_ref, bl_ref, wr_ref, w2_ref, b2_ref, o_ref):
    agg = (sp_ref[0] + sp_ref[1]) / jnp.maximum(d_ref[...], 1.0)
    h2 = jnp.maximum(
        jnp.dot(agg, wl_ref[...], preferred_element_type=jnp.float32)
        + bl_ref[...]
        + jnp.dot(h_ref[...], wr_ref[...], preferred_element_type=jnp.float32),
        0.0)
    o_ref[...] = (jnp.dot(h2, w2_ref[...], preferred_element_type=jnp.float32)
                  + b2_ref[...])


def _combinep(hp, sp, divp, Wlb, blb, Wrb, W2b, b2b):
    return pl.pallas_call(
        _combinep_body,
        grid=(PR // BLKP,),
        in_specs=[pl.BlockSpec((BLKP, 128), lambda i: (i, 0)),
                  pl.BlockSpec((NC, BLKP, 128), lambda i: (0, i, 0)),
                  pl.BlockSpec((BLKP, 128), lambda i: (i, 0)),
                  pl.BlockSpec((128, 256), lambda i: (0, 0)),
                  pl.BlockSpec((1, 256), lambda i: (0, 0)),
                  pl.BlockSpec((128, 256), lambda i: (0, 0)),
                  pl.BlockSpec((256, 256), lambda i: (0, 0)),
                  pl.BlockSpec((1, 256), lambda i: (0, 0))],
        out_specs=pl.BlockSpec((BLKP, 256), lambda i: (i, 0)),
        out_shape=jax.ShapeDtypeStruct((PR, 256), jnp.float32),
    )(hp, sp, divp, Wlb, blb, Wrb, W2b, b2b)


def _block_diag8(W):
    a, b = W.shape
    eye = jnp.eye(8, dtype=W.dtype)
    return (eye[:, None, :, None] * W[None, :, None, :]).reshape(8 * a, 8 * b)


def kernel(x, edge_index, W1, b1, Wl, bl, Wr, W2, b2):
    ei = edge_index.astype(jnp.int32)
    src = ei[0]
    dst = ei[1]

    W1b = _block_diag8(W1)
    b1b = jnp.tile(b1, 8).reshape(1, 128)
    Wlb = _block_diag8(Wl)
    blb = jnp.tile(bl, 8).reshape(1, 256)
    Wrb = _block_diag8(Wr)
    W2b = _block_diag8(W2)
    b2b = jnp.tile(b2, 8).reshape(1, 256)

    xp = jnp.pad(x.reshape(N // 8, 128), ((0, PR - N // 8), (0, 0)))
    hp = _lin1p(xp, W1b, b1b)
    summed, cnt = _sc_aggregate(hp.reshape(NP, D), src, dst)
    sp = summed.reshape(NC, PR, 128)
    cm = cnt[:NP] + cnt[NP:]
    expand = jnp.kron(jnp.eye(8, dtype=jnp.float32), jnp.ones((1, D), jnp.float32))
    divp = cm.reshape(PR, 8) @ expand
    outp = _combinep(hp, sp, divp, Wlb, blb, Wrb, W2b, b2b)
    return outp.reshape(NP, H)[:N]


# speedup vs baseline: 25.7436x; 1.0877x over previous
import jax
import jax.numpy as jnp
from jax import lax
from jax.experimental import pallas as pl
from jax.experimental.pallas import tpu as pltpu
from jax.experimental.pallas import tpu_sc as plsc

N = 100000
NP = 102400
PR = NP // 8
D = 16
H = 32
E = 1600000
NC = 2
NS = 16
EW = E // (NC * NS)
CHUNK = 400
NCHUNK = EW // CHUNK
IDXB = 5
ROWS_T = NP // NS
BLKP = 512


def _lin1p_body(x_ref, w_ref, b_ref, o_ref):
    o_ref[...] = jnp.maximum(
        jnp.dot(x_ref[...], w_ref[...], preferred_element_type=jnp.float32)
        + b_ref[...], 0.0)


def _lin1p(xp, W1b, b1b):
    return pl.pallas_call(
        _lin1p_body,
        grid=(PR // BLKP,),
        in_specs=[pl.BlockSpec((BLKP, 128), lambda i: (i, 0)),
                  pl.BlockSpec((128, 128), lambda i: (0, 0)),
                  pl.BlockSpec((1, 128), lambda i: (0, 0))],
        out_specs=pl.BlockSpec((BLKP, 128), lambda i: (i, 0)),
        out_shape=jax.ShapeDtypeStruct((PR, 128), jnp.float32),
    )(xp, W1b, b1b)


def _sc_body(h_hbm, src_hbm, dst_hbm, ones_hbm, z2d_hbm, z1d_hbm,
             sum_out, cnt_out,
             acc_sh, cnt_sh, srcb_v, dstb_v, rows_v0, rows_v1,
             ones_v, cb_v, sem0, sem1):
    c = lax.axis_index("c")
    s = lax.axis_index("s")
    zstart = s * ROWS_T
    pltpu.sync_copy(z2d_hbm, acc_sh.at[pl.ds(zstart, ROWS_T)])
    pltpu.sync_copy(z1d_hbm, cb_v)
    for off, ln in _SUBCHUNKS:
        pltpu.sync_copy(cb_v.at[pl.ds(0, ln)], cnt_sh.at[pl.ds(zstart + off, ln)])
    pltpu.sync_copy(ones_hbm, ones_v)
    plsc.subcore_barrier()

    rows_b = (rows_v0, rows_v1)
    sems = (sem0, sem1)

    def batch(b, carry):
        r0 = (c * NS + s) * NCHUNK + b * IDXB
        pltpu.sync_copy(src_hbm.at[pl.ds(r0, IDXB)], srcb_v)
        pltpu.sync_copy(dst_hbm.at[pl.ds(r0, IDXB)], dstb_v)
        for k in range(IDXB):
            pltpu.async_copy(h_hbm.at[srcb_v.at[k]], rows_b[k % 2], sems[k % 2])
            if k > 0:
                p = (k - 1) % 2
                pltpu.make_async_copy(h_hbm.at[srcb_v.at[k - 1]],
                                      rows_b[p], sems[p]).wait()
                pltpu.sync_copy(rows_b[p], acc_sh.at[dstb_v.at[k - 1]], add=True)
                pltpu.sync_copy(ones_v, cnt_sh.at[dstb_v.at[k - 1]], add=True)
        p = (IDXB - 1) % 2
        pltpu.make_async_copy(h_hbm.at[srcb_v.at[IDXB - 1]],
                              rows_b[p], sems[p]).wait()
        pltpu.sync_copy(rows_b[p], acc_sh.at[dstb_v.at[IDXB - 1]], add=True)
        pltpu.sync_copy(ones_v, cnt_sh.at[dstb_v.at[IDXB - 1]], add=True)
        return carry

    lax.fori_loop(0, NCHUNK // IDXB, batch, 0)
    plsc.subcore_barrier()
    pltpu.sync_copy(acc_sh.at[pl.ds(zstart, ROWS_T)],
                    sum_out.at[c, pl.ds(zstart, ROWS_T)])
    for off, ln in _SUBCHUNKS:
        pltpu.sync_copy(cnt_sh.at[pl.ds(zstart + off, ln)], cb_v.at[pl.ds(0, ln)])
        pltpu.sync_copy(cb_v.at[pl.ds(0, ln)],
                        cnt_out.at[pl.ds(c * NP + zstart + off, ln)])


CSTAGE = 800
_SUBCHUNKS = tuple((k * CSTAGE, CSTAGE) for k in range(ROWS_T // CSTAGE))


def _sc_aggregate(h_lin, src, dst):
    ones = jnp.ones((CHUNK,), jnp.float32)
    z2d = jnp.zeros((ROWS_T, D), jnp.float32)
    z1d = jnp.zeros((CSTAGE,), jnp.float32)
    mesh = plsc.VectorSubcoreMesh(core_axis_name="c", subcore_axis_name="s")
    f = pl.kernel(
        _sc_body,
        out_type=[jax.ShapeDtypeStruct((NC, NP, D), jnp.float32),
                  jax.ShapeDtypeStruct((NC * NP,), jnp.float32)],
        mesh=mesh,
        scratch_types=[
            pltpu.VMEM_SHARED((NP, D), jnp.float32),
            pltpu.VMEM_SHARED((NP,), jnp.float32),
            pltpu.VMEM((IDXB, CHUNK), jnp.int32),
            pltpu.VMEM((IDXB, CHUNK), jnp.int32),
            pltpu.VMEM((CHUNK, D), jnp.float32),
            pltpu.VMEM((CHUNK, D), jnp.float32),
            pltpu.VMEM((CHUNK,), jnp.float32),
            pltpu.VMEM((CSTAGE,), jnp.float32),
            pltpu.SemaphoreType.DMA,
            pltpu.SemaphoreType.DMA,
        ],
        compiler_params=pltpu.CompilerParams(use_tc_tiling_on_sc=False),
    )
    return f(h_lin, src.reshape(E // CHUNK, CHUNK), dst.reshape(E // CHUNK, CHUNK),
             ones, z2d, z1d)


def _combinep_body(h_ref, sp_ref, d_ref,
                   wl_ref, bl_ref, wr_ref, w2_ref, b2_ref, o_ref):
    agg = (sp_ref[0] + sp_ref[1]) / jnp.maximum(d_ref[...], 1.0)
    h2 = jnp.maximum(
        jnp.dot(agg, wl_ref[...], preferred_element_type=jnp.float32)
        + bl_ref[...]
        + jnp.dot(h_ref[...], wr_ref[...], preferred_element_type=jnp.float32),
        0.0)
    o_ref[...] = (jnp.dot(h2, w2_ref[...], preferred_element_type=jnp.float32)
                  + b2_ref[...])


def _combinep(hp, sp, divp, Wlb, blb, Wrb, W2b, b2b):
    return pl.pallas_call(
        _combinep_body,
        grid=(PR // BLKP,),
        in_specs=[pl.BlockSpec((BLKP, 128), lambda i: (i, 0)),
                  pl.BlockSpec((NC, BLKP, 128), lambda i: (0, i, 0)),
                  pl.BlockSpec((BLKP, 128), lambda i: (i, 0)),
                  pl.BlockSpec((128, 256), lambda i: (0, 0)),
                  pl.BlockSpec((1, 256), lambda i: (0, 0)),
                  pl.BlockSpec((128, 256), lambda i: (0, 0)),
                  pl.BlockSpec((256, 256), lambda i: (0, 0)),
                  pl.BlockSpec((1, 256), lambda i: (0, 0))],
        out_specs=pl.BlockSpec((BLKP, 256), lambda i: (i, 0)),
        out_shape=jax.ShapeDtypeStruct((PR, 256), jnp.float32),
    )(hp, sp, divp, Wlb, blb, Wrb, W2b, b2b)


def _block_diag8(W):
    a, b = W.shape
    eye = jnp.eye(8, dtype=W.dtype)
    return (eye[:, None, :, None] * W[None, :, None, :]).reshape(8 * a, 8 * b)


def kernel(x, edge_index, W1, b1, Wl, bl, Wr, W2, b2):
    ei = edge_index.astype(jnp.int32)
    src = ei[0]
    dst = ei[1]

    W1b = _block_diag8(W1)
    b1b = jnp.tile(b1, 8).reshape(1, 128)
    Wlb = _block_diag8(Wl)
    blb = jnp.tile(bl, 8).reshape(1, 256)
    Wrb = _block_diag8(Wr)
    W2b = _block_diag8(W2)
    b2b = jnp.tile(b2, 8).reshape(1, 256)

    xp = jnp.pad(x.reshape(N // 8, 128), ((0, PR - N // 8), (0, 0)))
    hp = _lin1p(xp, W1b, b1b)
    summed, cnt = _sc_aggregate(hp.reshape(NP, D), src, dst)
    sp = summed.reshape(NC, PR, 128)
    cm = cnt[:NP] + cnt[NP:]
    expand = jnp.kron(jnp.eye(8, dtype=jnp.float32), jnp.ones((1, D), jnp.float32))
    divp = cm.reshape(PR, 8) @ expand
    outp = _combinep(hp, sp, divp, Wlb, blb, Wrb, W2b, b2b)
    return outp.reshape(NP, H)[:N]


# speedup vs baseline: 26.6314x; 1.0345x over previous
import jax
import jax.numpy as jnp
from jax import lax
from jax.experimental import pallas as pl
from jax.experimental.pallas import tpu as pltpu
from jax.experimental.pallas import tpu_sc as plsc

N = 100000
NP = 102400
PR = NP // 8
PRX = N // 8
D = 16
H = 32
E = 1600000
NC = 2
NS = 16
EW = E // (NC * NS)
CHUNK = 400
NCHUNK = EW // CHUNK
IDXB = 5
ROWS_T = NP // NS
BLKP = 512


def _lin1p_body(x_ref, w_ref, b_ref, o_ref):
    o_ref[...] = jnp.maximum(
        jnp.dot(x_ref[...], w_ref[...], preferred_element_type=jnp.float32)
        + b_ref[...], 0.0)


def _lin1p(xp, W1b, b1b):
    return pl.pallas_call(
        _lin1p_body,
        grid=(pl.cdiv(PRX, BLKP),),
        in_specs=[pl.BlockSpec((BLKP, 128), lambda i: (i, 0)),
                  pl.BlockSpec((128, 128), lambda i: (0, 0)),
                  pl.BlockSpec((1, 128), lambda i: (0, 0))],
        out_specs=pl.BlockSpec((BLKP, 128), lambda i: (i, 0)),
        out_shape=jax.ShapeDtypeStruct((PRX, 128), jnp.float32),
    )(xp, W1b, b1b)


def _sc_body(h_hbm, src_hbm, dst_hbm, ones_hbm, z2d_hbm, z1d_hbm,
             sum_out, cnt_out,
             acc_sh, cnt_sh, srcb_v, dstb_v, rows_v0, rows_v1,
             ones_v, cb_v, sem0, sem1):
    c = lax.axis_index("c")
    s = lax.axis_index("s")
    zstart = s * ROWS_T
    pltpu.sync_copy(z2d_hbm, acc_sh.at[pl.ds(zstart, ROWS_T)])
    pltpu.sync_copy(z1d_hbm, cb_v)
    for off, ln in _SUBCHUNKS:
        pltpu.sync_copy(cb_v.at[pl.ds(0, ln)], cnt_sh.at[pl.ds(zstart + off, ln)])
    pltpu.sync_copy(ones_hbm, ones_v)
    plsc.subcore_barrier()

    rows_b = (rows_v0, rows_v1)
    sems = (sem0, sem1)

    def batch(b, carry):
        r0 = (c * NS + s) * NCHUNK + b * IDXB
        pltpu.sync_copy(src_hbm.at[pl.ds(r0, IDXB)], srcb_v)
        pltpu.sync_copy(dst_hbm.at[pl.ds(r0, IDXB)], dstb_v)
        for k in range(IDXB):
            pltpu.async_copy(h_hbm.at[srcb_v.at[k]], rows_b[k % 2], sems[k % 2])
            if k > 0:
                p = (k - 1) % 2
                pltpu.make_async_copy(h_hbm.at[srcb_v.at[k - 1]],
                                      rows_b[p], sems[p]).wait()
                pltpu.sync_copy(rows_b[p], acc_sh.at[dstb_v.at[k - 1]], add=True)
                pltpu.sync_copy(ones_v, cnt_sh.at[dstb_v.at[k - 1]], add=True)
        p = (IDXB - 1) % 2
        pltpu.make_async_copy(h_hbm.at[srcb_v.at[IDXB - 1]],
                              rows_b[p], sems[p]).wait()
        pltpu.sync_copy(rows_b[p], acc_sh.at[dstb_v.at[IDXB - 1]], add=True)
        pltpu.sync_copy(ones_v, cnt_sh.at[dstb_v.at[IDXB - 1]], add=True)
        return carry

    lax.fori_loop(0, NCHUNK // IDXB, batch, 0)
    plsc.subcore_barrier()
    pltpu.sync_copy(acc_sh.at[pl.ds(zstart, ROWS_T)],
                    sum_out.at[c, pl.ds(zstart, ROWS_T)])
    for off, ln in _SUBCHUNKS:
        pltpu.sync_copy(cnt_sh.at[pl.ds(zstart + off, ln)], cb_v.at[pl.ds(0, ln)])
        pltpu.sync_copy(cb_v.at[pl.ds(0, ln)],
                        cnt_out.at[pl.ds(c * NP + zstart + off, ln)])


CSTAGE = 800
_SUBCHUNKS = tuple((k * CSTAGE, CSTAGE) for k in range(ROWS_T // CSTAGE))


def _sc_aggregate(h_lin, src, dst):
    ones = jnp.ones((CHUNK,), jnp.float32)
    z2d = jnp.zeros((ROWS_T, D), jnp.float32)
    z1d = jnp.zeros((CSTAGE,), jnp.float32)
    mesh = plsc.VectorSubcoreMesh(core_axis_name="c", subcore_axis_name="s")
    f = pl.kernel(
        _sc_body,
        out_type=[jax.ShapeDtypeStruct((NC, NP, D), jnp.float32),
                  jax.ShapeDtypeStruct((NC * NP,), jnp.float32)],
        mesh=mesh,
        scratch_types=[
            pltpu.VMEM_SHARED((NP, D), jnp.float32),
            pltpu.VMEM_SHARED((NP,), jnp.float32),
            pltpu.VMEM((IDXB, CHUNK), jnp.int32),
            pltpu.VMEM((IDXB, CHUNK), jnp.int32),
            pltpu.VMEM((CHUNK, D), jnp.float32),
            pltpu.VMEM((CHUNK, D), jnp.float32),
            pltpu.VMEM((CHUNK,), jnp.float32),
            pltpu.VMEM((CSTAGE,), jnp.float32),
            pltpu.SemaphoreType.DMA,
            pltpu.SemaphoreType.DMA,
        ],
        compiler_params=pltpu.CompilerParams(use_tc_tiling_on_sc=False),
    )
    return f(h_lin, src.reshape(E // CHUNK, CHUNK), dst.reshape(E // CHUNK, CHUNK),
             ones, z2d, z1d)


def _combinep_body(h_ref, sp_ref, d_ref,
                   wl_ref, bl_ref, wr_ref, w2_ref, b2_ref, o_ref):
    agg = (sp_ref[0] + sp_ref[1]) / jnp.maximum(d_ref[...], 1.0)
    h2 = jnp.maximum(
        jnp.dot(agg, wl_ref[...], preferred_element_type=jnp.float32)
        + bl_ref[...]
        + jnp.dot(h_ref[...], wr_ref[...], preferred_element_type=jnp.float32),
        0.0)
    o_ref[...] = (jnp.dot(h2, w2_ref[...], preferred_element_type=jnp.float32)
                  + b2_ref[...])


def _combinep(hp, sp, divp, Wlb, blb, Wrb, W2b, b2b):
    return pl.pallas_call(
        _combinep_body,
        grid=(pl.cdiv(PRX, BLKP),),
        in_specs=[pl.BlockSpec((BLKP, 128), lambda i: (i, 0)),
                  pl.BlockSpec((NC, BLKP, 128), lambda i: (0, i, 0)),
                  pl.BlockSpec((BLKP, 128), lambda i: (i, 0)),
                  pl.BlockSpec((128, 256), lambda i: (0, 0)),
                  pl.BlockSpec((1, 256), lambda i: (0, 0)),
                  pl.BlockSpec((128, 256), lambda i: (0, 0)),
                  pl.BlockSpec((256, 256), lambda i: (0, 0)),
                  pl.BlockSpec((1, 256), lambda i: (0, 0))],
        out_specs=pl.BlockSpec((BLKP, 256), lambda i: (i, 0)),
        out_shape=jax.ShapeDtypeStruct((PRX, 256), jnp.float32),
    )(hp, sp, divp, Wlb, blb, Wrb, W2b, b2b)


def _block_diag8(W):
    a, b = W.shape
    eye = jnp.eye(8, dtype=W.dtype)
    return (eye[:, None, :, None] * W[None, :, None, :]).reshape(8 * a, 8 * b)


def kernel(x, edge_index, W1, b1, Wl, bl, Wr, W2, b2):
    ei = edge_index.astype(jnp.int32)
    src = ei[0]
    dst = ei[1]

    W1b = _block_diag8(W1)
    b1b = jnp.tile(b1, 8).reshape(1, 128)
    Wlb = _block_diag8(Wl)
    blb = jnp.tile(bl, 8).reshape(1, 256)
    Wrb = _block_diag8(Wr)
    W2b = _block_diag8(W2)
    b2b = jnp.tile(b2, 8).reshape(1, 256)

    xp = x.reshape(PRX, 128)
    hp = _lin1p(xp, W1b, b1b)
    summed, cnt = _sc_aggregate(hp.reshape(N, D), src, dst)
    sp = summed.reshape(NC, PR, 128)
    cm = cnt[:N] + cnt[NP:NP + N]
    expand = jnp.kron(jnp.eye(8, dtype=jnp.float32), jnp.ones((1, D), jnp.float32))
    divp = cm.reshape(PRX, 8) @ expand
    outp = _combinep(hp, sp, divp, Wlb, blb, Wrb, W2b, b2b)
    return outp.reshape(N, H)
